# baseline (speedup 1.0000x reference)
import functools

import jax
import jax.numpy as jnp
from jax import lax
from jax.experimental import pallas as pl
from jax.experimental.pallas import tpu as pltpu
from jax.experimental.pallas import tpu_sc as plsc

N = 10000
E = 320000
D = 128
H = 8
HD = 16
EPS = 1e-5

NC = 2
NS = 16
H_SC = H // NC
QW = D // NC
KVW = D
ACC_W = 72
EPT = E // NS
CHUNK = 80
NCHUNK = EPT // CHUNK
PAIRS = NCHUNK // 2
ROWS_PER_TILE = 632


def _proj_body(x_ref, wq0, wq1, wkv0, wkv1, bq0, bq1, bkv0, bkv1, q_ref, kv_ref):
    x = x_ref[...]
    f32 = jnp.float32
    q_ref[0] = jnp.dot(x, wq0[...], preferred_element_type=f32) + bq0[...]
    q_ref[1] = jnp.dot(x, wq1[...], preferred_element_type=f32) + bq1[...]
    kv_ref[0] = jnp.dot(x, wkv0[...], preferred_element_type=f32) + bkv0[...]
    kv_ref[1] = jnp.dot(x, wkv1[...], preferred_element_type=f32) + bkv1[...]


def _proj(x, wq0, wq1, wkv0, wkv1, bq0, bq1, bkv0, bkv1):
    blk = 400
    grid = N // blk
    return pl.pallas_call(
        _proj_body,
        grid=(grid,),
        in_specs=[
            pl.BlockSpec((blk, D), lambda i: (i, 0)),
            pl.BlockSpec((D, QW), lambda i: (0, 0)),
            pl.BlockSpec((D, QW), lambda i: (0, 0)),
            pl.BlockSpec((D, KVW), lambda i: (0, 0)),
            pl.BlockSpec((D, KVW), lambda i: (0, 0)),
            pl.BlockSpec((1, QW), lambda i: (0, 0)),
            pl.BlockSpec((1, QW), lambda i: (0, 0)),
            pl.BlockSpec((1, KVW), lambda i: (0, 0)),
            pl.BlockSpec((1, KVW), lambda i: (0, 0)),
        ],
        out_specs=[
            pl.BlockSpec((NC, blk, QW), lambda i: (0, i, 0)),
            pl.BlockSpec((NC, blk, KVW), lambda i: (0, i, 0)),
        ],
        out_shape=[
            jax.ShapeDtypeStruct((NC, N, QW), jnp.float32),
            jax.ShapeDtypeStruct((NC, N, KVW), jnp.float32),
        ],
    )(x, wq0, wq1, wkv0, wkv1, bq0, bq1, bkv0, bkv1)


@functools.partial(
    pl.kernel,
    out_type=jax.ShapeDtypeStruct((NC, N, ACC_W), jnp.float32),
    mesh=plsc.VectorSubcoreMesh(core_axis_name="c", subcore_axis_name="s"),
    compiler_params=pltpu.CompilerParams(
        use_tc_tiling_on_sc=False, needs_layout_passes=False),
    scratch_types=[
        pltpu.VMEM((CHUNK,), jnp.int32),
        pltpu.VMEM((CHUNK,), jnp.int32),
        pltpu.VMEM((CHUNK,), jnp.int32),
        pltpu.VMEM((CHUNK,), jnp.int32),
        pltpu.VMEM((CHUNK,), jnp.int32),
        pltpu.VMEM((CHUNK,), jnp.int32),
        pltpu.VMEM((CHUNK, QW), jnp.float32),
        pltpu.VMEM((CHUNK, QW), jnp.float32),
        pltpu.VMEM((CHUNK, KVW), jnp.float32),
        pltpu.VMEM((CHUNK, KVW), jnp.float32),
        pltpu.VMEM((CHUNK, ACC_W), jnp.float32),
        pltpu.VMEM((CHUNK, ACC_W), jnp.float32),
        pltpu.VMEM_SHARED((N, ACC_W), jnp.float32),
        pltpu.SemaphoreType.DMA,
        pltpu.SemaphoreType.DMA,
        pltpu.SemaphoreType.DMA,
        pltpu.SemaphoreType.DMA,
        pltpu.SemaphoreType.DMA,
        pltpu.SemaphoreType.DMA,
        pltpu.SemaphoreType.DMA,
        pltpu.SemaphoreType.DMA,
        pltpu.SemaphoreType.DMA,
        pltpu.SemaphoreType.DMA,
    ],
)
def _edge_kernel(q_hbm, kv_hbm, src_hbm, dst_hbm, zero_hbm, out_hbm,
                 gidx0, gidx1, gdidx0, gdidx1, sidx0, sidx1,
                 q2a, q2b, kv2a, kv2b, w2a, w2b, accum,
                 semq0, semq1, semk0, semk1, semw0, semw1,
                 semsi0, semsi1, semdi0, semdi1):
    semq = (semq0, semq1)
    semk = (semk0, semk1)
    semw = (semw0, semw1)
    semsi = (semsi0, semsi1)
    semdi = (semdi0, semdi1)
    gidx = (gidx0, gidx1)
    gdidx = (gdidx0, gdidx1)
    sidx = (sidx0, sidx1)
    q2 = (q2a, q2b)
    kv2 = (kv2a, kv2b)
    w2 = (w2a, w2b)
    c = lax.axis_index("c")
    s = lax.axis_index("s")
    lane = lax.iota(jnp.int32, 16)
    e0 = pl.multiple_of(s * EPT, 8)

    def _copy_buf(dst_ref, src_ref):
        for g in range(CHUNK // 16):
            dst_ref[pl.ds(g * 16, 16)] = src_ref[pl.ds(g * 16, 16)]

    r0 = pl.multiple_of(jnp.minimum(s * ROWS_PER_TILE, N - ROWS_PER_TILE), 8)
    pltpu.sync_copy(zero_hbm.at[pl.ds(r0, ROWS_PER_TILE)],
                    accum.at[pl.ds(r0, ROWS_PER_TILE)])
    pltpu.sync_copy(zero_hbm.at[pl.ds(0, CHUNK)], w2[0])
    pltpu.sync_copy(zero_hbm.at[pl.ds(0, CHUNK)], w2[1])
    for b in range(2):
        off = pl.multiple_of(e0 + b * CHUNK, 8)
        pltpu.sync_copy(src_hbm.at[pl.ds(off, CHUNK)], gidx[b])
        pltpu.sync_copy(dst_hbm.at[pl.ds(off, CHUNK)], gdidx[b])
        _copy_buf(sidx[b], gidx[b])
    plsc.subcore_barrier()
    pltpu.async_copy(w2[0], accum.at[sidx[0]], semw[0], add=True)
    pltpu.async_copy(w2[1], accum.at[sidx[1]], semw[1], add=True)
    pltpu.async_copy(q_hbm.at[c].at[gidx[0]], q2[0], semq[0])
    pltpu.async_copy(kv_hbm.at[c].at[gdidx[0]], kv2[0], semk[0])
    pltpu.async_copy(q_hbm.at[c].at[gidx[1]], q2[1], semq[1])
    pltpu.async_copy(kv_hbm.at[c].at[gdidx[1]], kv2[1], semk[1])

    def _compute(b):
        q_v, kv_v, w_v = q2[b], kv2[b], w2[b]

        @plsc.parallel_loop(0, CHUNK // 16)
        def _group(g):
            rows = g * 16 + lane
            for h0 in range(0, H_SC, 4):
                hp = (h0, h0 + 1, h0 + 2, h0 + 3)
                parts = {h: [jnp.zeros((16,), jnp.float32) for _ in range(2)]
                         for h in hp}
                for d in range(HD):
                    rot_d = jnp.bitwise_and(lane + d, HD - 1)
                    for h in hp:
                        col = h * HD + rot_d
                        qv = plsc.load_gather(q_v, [rows, col])
                        kv = plsc.load_gather(kv_v, [rows, col])
                        parts[h][d % 2] = parts[h][d % 2] + qv * kv
                ex = {h: jnp.exp((parts[h][0] + parts[h][1]) * 0.25)
                      for h in hp}
                for d in range(HD):
                    rot_d = jnp.bitwise_and(lane + d, HD - 1)
                    for h in hp:
                        colw = h * HD + rot_d
                        vv = plsc.load_gather(kv_v, [rows, QW + colw])
                        plsc.store_scatter(w_v, [rows, colw], ex[h] * vv)
                for h in hp:
                    cole = jnp.full((16,), QW + h, jnp.int32)
                    plsc.store_scatter(w_v, [rows, cole], ex[h])

    def _process(b, i, p):
        pltpu.make_async_copy(q_hbm.at[c].at[gidx[b]], q2[b], semq[b]).wait()
        pltpu.make_async_copy(kv_hbm.at[c].at[gdidx[b]], kv2[b], semk[b]).wait()
        pltpu.make_async_copy(w2[b], accum.at[sidx[b]], semw[b]).wait()
        _copy_buf(sidx[b], gidx[b])

        @pl.when(p < PAIRS - 1)
        def _():
            off = pl.multiple_of(e0 + (i + 2) * CHUNK, 8)
            pltpu.async_copy(src_hbm.at[pl.ds(off, CHUNK)], gidx[b], semsi[b])
            pltpu.async_copy(dst_hbm.at[pl.ds(off, CHUNK)], gdidx[b], semdi[b])

        _compute(b)
        pltpu.async_copy(w2[b], accum.at[sidx[b]], semw[b], add=True)

        @pl.when(p < PAIRS - 1)
        def _():
            off = pl.multiple_of(e0 + (i + 2) * CHUNK, 8)
            pltpu.make_async_copy(src_hbm.at[pl.ds(off, CHUNK)], gidx[b], semsi[b]).wait()
            pltpu.make_async_copy(dst_hbm.at[pl.ds(off, CHUNK)], gdidx[b], semdi[b]).wait()
            pltpu.async_copy(q_hbm.at[c].at[gidx[b]], q2[b], semq[b])
            pltpu.async_copy(kv_hbm.at[c].at[gdidx[b]], kv2[b], semk[b])

    @pl.loop(0, PAIRS)
    def _pair(p):
        _process(0, 2 * p, p)
        _process(1, 2 * p + 1, p)

    pltpu.make_async_copy(w2[0], accum.at[sidx[0]], semw[0]).wait()
    pltpu.make_async_copy(w2[1], accum.at[sidx[1]], semw[1]).wait()
    plsc.subcore_barrier()
    pltpu.sync_copy(accum.at[pl.ds(r0, ROWS_PER_TILE)],
                    out_hbm.at[c, pl.ds(r0, ROWS_PER_TILE)])


def _finish_body(p_ref, s_ref, w_ref, b_ref, l1w_ref, l1b_ref, l2w_ref, l2b_ref, o_ref):
    p0 = p_ref[0]
    p1 = p_ref[1]
    numer = jnp.concatenate([p0[:, :QW], p1[:, :QW]], axis=1)
    denom = jnp.concatenate([p0[:, QW:QW + H_SC], p1[:, QW:QW + H_SC]], axis=1)
    recip = 1.0 / (denom + 1e-16)
    attn = numer * jnp.dot(recip, s_ref[...], preferred_element_type=jnp.float32)

    mu = jnp.mean(attn, axis=-1, keepdims=True)
    var = jnp.mean((attn - mu) ** 2, axis=-1, keepdims=True)
    y = (attn - mu) / jnp.sqrt(var + EPS) * l1w_ref[...] + l1b_ref[...]

    out = jnp.dot(y, w_ref[...], preferred_element_type=jnp.float32) + b_ref[...]
    mu2 = jnp.mean(out, axis=-1, keepdims=True)
    var2 = jnp.mean((out - mu2) ** 2, axis=-1, keepdims=True)
    o_ref[...] = (out - mu2) / jnp.sqrt(var2 + EPS) * l2w_ref[...] + l2b_ref[...]


def _finish(partial, S, Wout, bout, l1w, l1b, l2w, l2b):
    blk = 400
    grid = N // blk
    return pl.pallas_call(
        _finish_body,
        grid=(grid,),
        in_specs=[
            pl.BlockSpec((NC, blk, ACC_W), lambda i: (0, i, 0)),
            pl.BlockSpec((H, D), lambda i: (0, 0)),
            pl.BlockSpec((D, D), lambda i: (0, 0)),
            pl.BlockSpec((1, D), lambda i: (0, 0)),
            pl.BlockSpec((1, D), lambda i: (0, 0)),
            pl.BlockSpec((1, D), lambda i: (0, 0)),
            pl.BlockSpec((1, D), lambda i: (0, 0)),
            pl.BlockSpec((1, D), lambda i: (0, 0)),
        ],
        out_specs=pl.BlockSpec((blk, D), lambda i: (i, 0)),
        out_shape=jax.ShapeDtypeStruct((N, D), jnp.float32),
    )(partial, S, Wout, bout, l1w, l1b, l2w, l2b)


def kernel(x, edge_index, WQ, bQ, WK, bK, WV, bV, Wout, bout, ln1_w, ln1_b, ln2_w, ln2_b):
    ei = edge_index.astype(jnp.int32)
    src = ei[0]
    dst = ei[1]
    wq0, wq1 = WQ[:, :QW], WQ[:, QW:]
    wkv0 = jnp.concatenate([WK[:, :QW], WV[:, :QW]], axis=1)
    wkv1 = jnp.concatenate([WK[:, QW:], WV[:, QW:]], axis=1)
    bq0, bq1 = bQ[:QW].reshape(1, QW), bQ[QW:].reshape(1, QW)
    bkv0 = jnp.concatenate([bK[:QW], bV[:QW]]).reshape(1, KVW)
    bkv1 = jnp.concatenate([bK[QW:], bV[QW:]]).reshape(1, KVW)
    q_tab, kv_tab = _proj(x, wq0, wq1, wkv0, wkv1, bq0, bq1, bkv0, bkv1)
    zeros = jnp.zeros((N, ACC_W), jnp.float32)
    partial = _edge_kernel(q_tab, kv_tab, src, dst, zeros)
    S = jnp.repeat(jnp.eye(H, dtype=jnp.float32), HD, axis=1)
    return _finish(partial, S, Wout, bout.reshape(1, D),
                   ln1_w.reshape(1, D), ln1_b.reshape(1, D),
                   ln2_w.reshape(1, D), ln2_b.reshape(1, D))


# speedup vs baseline: 1.4902x; 1.4902x over previous
import functools

import jax
import jax.numpy as jnp
from jax import lax
from jax.experimental import pallas as pl
from jax.experimental.pallas import tpu as pltpu
from jax.experimental.pallas import tpu_sc as plsc

N = 10000
E = 320000
D = 128
H = 8
HD = 16
EPS = 1e-5

NC = 2
NS = 16
H_SC = H // NC
QW = D // NC
KVW = D
ACC_W = 72
EPT = E // NS
CHUNK = 80
NCHUNK = EPT // CHUNK
PAIRS = NCHUNK // 2
ROWS_PER_TILE = 632


def _proj_body(x_ref, wq0, wq1, wkv0, wkv1, bq0, bq1, bkv0, bkv1, q_ref, kv_ref):
    x = x_ref[...]
    f32 = jnp.float32
    q_ref[0] = jnp.dot(x, wq0[...], preferred_element_type=f32) + bq0[...]
    q_ref[1] = jnp.dot(x, wq1[...], preferred_element_type=f32) + bq1[...]
    kv_ref[0] = jnp.dot(x, wkv0[...], preferred_element_type=f32) + bkv0[...]
    kv_ref[1] = jnp.dot(x, wkv1[...], preferred_element_type=f32) + bkv1[...]


def _proj(x, wq0, wq1, wkv0, wkv1, bq0, bq1, bkv0, bkv1):
    blk = 400
    grid = N // blk
    return pl.pallas_call(
        _proj_body,
        grid=(grid,),
        in_specs=[
            pl.BlockSpec((blk, D), lambda i: (i, 0)),
            pl.BlockSpec((D, QW), lambda i: (0, 0)),
            pl.BlockSpec((D, QW), lambda i: (0, 0)),
            pl.BlockSpec((D, KVW), lambda i: (0, 0)),
            pl.BlockSpec((D, KVW), lambda i: (0, 0)),
            pl.BlockSpec((1, QW), lambda i: (0, 0)),
            pl.BlockSpec((1, QW), lambda i: (0, 0)),
            pl.BlockSpec((1, KVW), lambda i: (0, 0)),
            pl.BlockSpec((1, KVW), lambda i: (0, 0)),
        ],
        out_specs=[
            pl.BlockSpec((NC, blk, QW), lambda i: (0, i, 0)),
            pl.BlockSpec((NC, blk, KVW), lambda i: (0, i, 0)),
        ],
        out_shape=[
            jax.ShapeDtypeStruct((NC, N, QW), jnp.float32),
            jax.ShapeDtypeStruct((NC, N, KVW), jnp.float32),
        ],
    )(x, wq0, wq1, wkv0, wkv1, bq0, bq1, bkv0, bkv1)


@functools.partial(
    pl.kernel,
    out_type=jax.ShapeDtypeStruct((NC, N, ACC_W), jnp.float32),
    mesh=plsc.VectorSubcoreMesh(core_axis_name="c", subcore_axis_name="s"),
    compiler_params=pltpu.CompilerParams(
        use_tc_tiling_on_sc=False, needs_layout_passes=False),
    scratch_types=[
        pltpu.VMEM((CHUNK,), jnp.int32),
        pltpu.VMEM((CHUNK,), jnp.int32),
        pltpu.VMEM((CHUNK,), jnp.int32),
        pltpu.VMEM((CHUNK,), jnp.int32),
        pltpu.VMEM((CHUNK,), jnp.int32),
        pltpu.VMEM((CHUNK,), jnp.int32),
        pltpu.VMEM((CHUNK, QW), jnp.float32),
        pltpu.VMEM((CHUNK, QW), jnp.float32),
        pltpu.VMEM((CHUNK, KVW), jnp.float32),
        pltpu.VMEM((CHUNK, KVW), jnp.float32),
        pltpu.VMEM((CHUNK, ACC_W), jnp.float32),
        pltpu.VMEM((CHUNK, ACC_W), jnp.float32),
        pltpu.VMEM_SHARED((N, ACC_W), jnp.float32),
        pltpu.SemaphoreType.DMA,
        pltpu.SemaphoreType.DMA,
        pltpu.SemaphoreType.DMA,
        pltpu.SemaphoreType.DMA,
        pltpu.SemaphoreType.DMA,
        pltpu.SemaphoreType.DMA,
        pltpu.SemaphoreType.DMA,
        pltpu.SemaphoreType.DMA,
        pltpu.SemaphoreType.DMA,
        pltpu.SemaphoreType.DMA,
    ],
)
def _edge_kernel(q_hbm, kv_hbm, src_hbm, dst_hbm, zero_hbm, out_hbm,
                 gidx0, gidx1, gdidx0, gdidx1, sidx0, sidx1,
                 q2a, q2b, kv2a, kv2b, w2a, w2b, accum,
                 semq0, semq1, semk0, semk1, semw0, semw1,
                 semsi0, semsi1, semdi0, semdi1):
    semq = (semq0, semq1)
    semk = (semk0, semk1)
    semw = (semw0, semw1)
    semsi = (semsi0, semsi1)
    semdi = (semdi0, semdi1)
    gidx = (gidx0, gidx1)
    gdidx = (gdidx0, gdidx1)
    sidx = (sidx0, sidx1)
    q2 = (q2a, q2b)
    kv2 = (kv2a, kv2b)
    w2 = (w2a, w2b)
    c = lax.axis_index("c")
    s = lax.axis_index("s")
    lane = lax.iota(jnp.int32, 16)
    e0 = pl.multiple_of(s * EPT, 8)

    def _copy_buf(dst_ref, src_ref):
        for g in range(CHUNK // 16):
            dst_ref[pl.ds(g * 16, 16)] = src_ref[pl.ds(g * 16, 16)]

    r0 = pl.multiple_of(jnp.minimum(s * ROWS_PER_TILE, N - ROWS_PER_TILE), 8)
    pltpu.sync_copy(zero_hbm.at[pl.ds(r0, ROWS_PER_TILE)],
                    accum.at[pl.ds(r0, ROWS_PER_TILE)])
    pltpu.sync_copy(zero_hbm.at[pl.ds(0, CHUNK)], w2[0])
    pltpu.sync_copy(zero_hbm.at[pl.ds(0, CHUNK)], w2[1])
    for b in range(2):
        off = pl.multiple_of(e0 + b * CHUNK, 8)
        pltpu.sync_copy(src_hbm.at[pl.ds(off, CHUNK)], gidx[b])
        pltpu.sync_copy(dst_hbm.at[pl.ds(off, CHUNK)], gdidx[b])
        _copy_buf(sidx[b], gidx[b])
    plsc.subcore_barrier()
    pltpu.async_copy(w2[0], accum.at[sidx[0]], semw[0], add=True)
    pltpu.async_copy(w2[1], accum.at[sidx[1]], semw[1], add=True)
    pltpu.async_copy(q_hbm.at[c].at[gidx[0]], q2[0], semq[0])
    pltpu.async_copy(kv_hbm.at[c].at[gdidx[0]], kv2[0], semk[0])
    pltpu.async_copy(q_hbm.at[c].at[gidx[1]], q2[1], semq[1])
    pltpu.async_copy(kv_hbm.at[c].at[gdidx[1]], kv2[1], semk[1])

    def _compute(b):
        q_v, kv_v, w_v = q2[b], kv2[b], w2[b]

        @plsc.parallel_loop(0, CHUNK // 16)
        def _group(g):
            rows = g * 16 + lane
            for h0 in range(0, H_SC, 2):
                hp = (h0, h0 + 1)
                parts = {h: [jnp.zeros((16,), jnp.float32) for _ in range(2)]
                         for h in hp}
                for d in range(HD):
                    rot_d = jnp.bitwise_and(lane + d, HD - 1)
                    for h in hp:
                        col = h * HD + rot_d
                        qv = plsc.load_gather(q_v, [rows, col])
                        kv = plsc.load_gather(kv_v, [rows, col])
                        parts[h][d % 2] = parts[h][d % 2] + qv * kv
                ex = {h: jnp.exp((parts[h][0] + parts[h][1]) * 0.25)
                      for h in hp}
                for d in range(HD):
                    rot_d = jnp.bitwise_and(lane + d, HD - 1)
                    for h in hp:
                        colw = h * HD + rot_d
                        vv = plsc.load_gather(kv_v, [rows, QW + colw])
                        plsc.store_scatter(w_v, [rows, colw], ex[h] * vv)
                for h in hp:
                    cole = jnp.full((16,), QW + h, jnp.int32)
                    plsc.store_scatter(w_v, [rows, cole], ex[h])

    def _process(b, i, p):
        pltpu.make_async_copy(q_hbm.at[c].at[gidx[b]], q2[b], semq[b]).wait()
        pltpu.make_async_copy(kv_hbm.at[c].at[gdidx[b]], kv2[b], semk[b]).wait()
        pltpu.make_async_copy(w2[b], accum.at[sidx[b]], semw[b]).wait()
        _copy_buf(sidx[b], gidx[b])

        @pl.when(p < PAIRS - 1)
        def _():
            off = pl.multiple_of(e0 + (i + 2) * CHUNK, 8)
            pltpu.async_copy(src_hbm.at[pl.ds(off, CHUNK)], gidx[b], semsi[b])
            pltpu.async_copy(dst_hbm.at[pl.ds(off, CHUNK)], gdidx[b], semdi[b])

        _compute(b)
        pltpu.async_copy(w2[b], accum.at[sidx[b]], semw[b], add=True)

        @pl.when(p < PAIRS - 1)
        def _():
            off = pl.multiple_of(e0 + (i + 2) * CHUNK, 8)
            pltpu.make_async_copy(src_hbm.at[pl.ds(off, CHUNK)], gidx[b], semsi[b]).wait()
            pltpu.make_async_copy(dst_hbm.at[pl.ds(off, CHUNK)], gdidx[b], semdi[b]).wait()
            pltpu.async_copy(q_hbm.at[c].at[gidx[b]], q2[b], semq[b])
            pltpu.async_copy(kv_hbm.at[c].at[gdidx[b]], kv2[b], semk[b])

    @pl.loop(0, PAIRS)
    def _pair(p):
        _process(0, 2 * p, p)
        _process(1, 2 * p + 1, p)

    pltpu.make_async_copy(w2[0], accum.at[sidx[0]], semw[0]).wait()
    pltpu.make_async_copy(w2[1], accum.at[sidx[1]], semw[1]).wait()
    plsc.subcore_barrier()
    pltpu.sync_copy(accum.at[pl.ds(r0, ROWS_PER_TILE)],
                    out_hbm.at[c, pl.ds(r0, ROWS_PER_TILE)])


def _finish_body(p_ref, s_ref, w_ref, b_ref, l1w_ref, l1b_ref, l2w_ref, l2b_ref, o_ref):
    p0 = p_ref[0]
    p1 = p_ref[1]
    numer = jnp.concatenate([p0[:, :QW], p1[:, :QW]], axis=1)
    denom = jnp.concatenate([p0[:, QW:QW + H_SC], p1[:, QW:QW + H_SC]], axis=1)
    recip = 1.0 / (denom + 1e-16)
    attn = numer * jnp.dot(recip, s_ref[...], preferred_element_type=jnp.float32)

    mu = jnp.mean(attn, axis=-1, keepdims=True)
    var = jnp.mean((attn - mu) ** 2, axis=-1, keepdims=True)
    y = (attn - mu) / jnp.sqrt(var + EPS) * l1w_ref[...] + l1b_ref[...]

    out = jnp.dot(y, w_ref[...], preferred_element_type=jnp.float32) + b_ref[...]
    mu2 = jnp.mean(out, axis=-1, keepdims=True)
    var2 = jnp.mean((out - mu2) ** 2, axis=-1, keepdims=True)
    o_ref[...] = (out - mu2) / jnp.sqrt(var2 + EPS) * l2w_ref[...] + l2b_ref[...]


def _finish(partial, S, Wout, bout, l1w, l1b, l2w, l2b):
    blk = 400
    grid = N // blk
    return pl.pallas_call(
        _finish_body,
        grid=(grid,),
        in_specs=[
            pl.BlockSpec((NC, blk, ACC_W), lambda i: (0, i, 0)),
            pl.BlockSpec((H, D), lambda i: (0, 0)),
            pl.BlockSpec((D, D), lambda i: (0, 0)),
            pl.BlockSpec((1, D), lambda i: (0, 0)),
            pl.BlockSpec((1, D), lambda i: (0, 0)),
            pl.BlockSpec((1, D), lambda i: (0, 0)),
            pl.BlockSpec((1, D), lambda i: (0, 0)),
            pl.BlockSpec((1, D), lambda i: (0, 0)),
        ],
        out_specs=pl.BlockSpec((blk, D), lambda i: (i, 0)),
        out_shape=jax.ShapeDtypeStruct((N, D), jnp.float32),
    )(partial, S, Wout, bout, l1w, l1b, l2w, l2b)


def kernel(x, edge_index, WQ, bQ, WK, bK, WV, bV, Wout, bout, ln1_w, ln1_b, ln2_w, ln2_b):
    ei = edge_index.astype(jnp.int32)
    src = ei[0]
    dst = ei[1]
    wq0, wq1 = WQ[:, :QW], WQ[:, QW:]
    wkv0 = jnp.concatenate([WK[:, :QW], WV[:, :QW]], axis=1)
    wkv1 = jnp.concatenate([WK[:, QW:], WV[:, QW:]], axis=1)
    bq0, bq1 = bQ[:QW].reshape(1, QW), bQ[QW:].reshape(1, QW)
    bkv0 = jnp.concatenate([bK[:QW], bV[:QW]]).reshape(1, KVW)
    bkv1 = jnp.concatenate([bK[QW:], bV[QW:]]).reshape(1, KVW)
    q_tab, kv_tab = _proj(x, wq0, wq1, wkv0, wkv1, bq0, bq1, bkv0, bkv1)
    zeros = jnp.zeros((N, ACC_W), jnp.float32)
    partial = _edge_kernel(q_tab, kv_tab, src, dst, zeros)
    S = jnp.repeat(jnp.eye(H, dtype=jnp.float32), HD, axis=1)
    return _finish(partial, S, Wout, bout.reshape(1, D),
                   ln1_w.reshape(1, D), ln1_b.reshape(1, D),
                   ln2_w.reshape(1, D), ln2_b.reshape(1, D))
